# Initial kernel scaffold; baseline (speedup 1.0000x reference)
import jax
import jax.numpy as jnp
from jax.experimental import pallas as pl


def kernel(input, weight):
    raise NotImplementedError("write your pallas kernel here")


# speedup vs baseline: 1.9607x; 1.9607x over previous
import functools

import jax
import jax.numpy as jnp
from jax import lax
from jax.experimental import pallas as pl
from jax.experimental.pallas import tpu as pltpu
from jax.experimental.pallas import tpu_sc as plsc

_NUM_CORES = 2
_NUM_SUBCORES = 16
_NW = _NUM_CORES * _NUM_SUBCORES
_LANES = 16
_NBUF = 3
_CH = 128


@functools.partial(jax.jit, static_argnames=("B", "F", "D"))
def _embedding_bag(idx3, weight, *, B, F, D):
    nb = B // _NW
    nch = nb // _CH

    mesh = plsc.VectorSubcoreMesh(core_axis_name="c", subcore_axis_name="s")

    @functools.partial(
        pl.kernel,
        out_type=jax.ShapeDtypeStruct((B, D), jnp.float32),
        mesh=mesh,
        compiler_params=pltpu.CompilerParams(use_tc_tiling_on_sc=False),
        scratch_types=[
            pltpu.VMEM((F * nch, _CH), jnp.int32),
            pltpu.VMEM((nb, D), jnp.float32),
            [pltpu.VMEM((nb, D), jnp.float32) for _ in range(_NBUF)],
            [pltpu.SemaphoreType.DMA for _ in range(_NBUF)],
            pltpu.SemaphoreType.DMA,
        ],
    )
    def run(idx_hbm, w_hbm, out_hbm, idx_v, acc_v, bufs, sems, sem_a):
        wid = lax.axis_index("s") * _NUM_CORES + lax.axis_index("c")
        base = wid * nb
        pltpu.sync_copy(idx_hbm.at[wid], idx_v)

        def gather_field(f, dst, sem):
            return [
                pltpu.async_copy(
                    w_hbm.at[idx_v.at[f * nch + k]], dst.at[pl.ds(k * _CH, _CH)], sem
                )
                for k in range(nch)
            ]

        cp_a = gather_field(0, acc_v, sem_a)
        cps = [None] * _NBUF
        for f in range(1, min(1 + _NBUF, F)):
            p = (f - 1) % _NBUF
            cps[p] = gather_field(f, bufs[p], sems[p])
        for cp in cp_a:
            cp.wait()

        for f in range(1, F):
            p = (f - 1) % _NBUF
            for cp in cps[p]:
                cp.wait()
            buf = bufs[p]

            @plsc.parallel_loop(0, nb, 1, unroll=8)
            def body(i, buf=buf):
                for h in range(0, D, _LANES):
                    plsc.addupdate(
                        acc_v.at[i, pl.ds(h, _LANES)], buf[i, pl.ds(h, _LANES)]
                    )

            nxt = f + _NBUF
            if nxt < F:
                cps[p] = gather_field(nxt, bufs[p], sems[p])

        pltpu.sync_copy(acc_v, out_hbm.at[pl.ds(base, nb)])

    return run(idx3, weight)


def kernel(input, weight):
    B, F = input.shape
    _, D = weight.shape
    nch = B // _NW // _CH
    idx3 = (
        input.T.reshape(F, _NW, nch, _CH)
        .transpose(1, 0, 2, 3)
        .reshape(_NW, F * nch, _CH)
    )
    return _embedding_bag(idx3, weight, B=B, F=F, D=D)


# speedup vs baseline: 1.9950x; 1.0175x over previous
import functools

import jax
import jax.numpy as jnp
from jax import lax
from jax.experimental import pallas as pl
from jax.experimental.pallas import tpu as pltpu
from jax.experimental.pallas import tpu_sc as plsc

_NUM_CORES = 2
_NUM_SUBCORES = 16
_NW = _NUM_CORES * _NUM_SUBCORES
_LANES = 16
_NBUF = 3
_KB = 8


@functools.partial(jax.jit, static_argnames=("B", "F", "D"))
def _embedding_bag(idx2, weight, *, B, F, D):
    nb = B // _NW
    rpc = 128 // F
    cl = rpc * F
    nch = nb // rpc
    ns = nch // _KB
    rows_ps = _KB * cl
    out_ps = _KB * rpc

    mesh = plsc.VectorSubcoreMesh(core_axis_name="c", subcore_axis_name="s")

    @functools.partial(
        pl.kernel,
        out_type=jax.ShapeDtypeStruct((B, D), jnp.float32),
        mesh=mesh,
        compiler_params=pltpu.CompilerParams(use_tc_tiling_on_sc=False),
        scratch_types=[
            pltpu.VMEM((nch, cl), jnp.int32),
            pltpu.VMEM((nb, D), jnp.float32),
            [pltpu.VMEM((rows_ps, D), jnp.float32) for _ in range(_NBUF)],
            [pltpu.SemaphoreType.DMA for _ in range(_NBUF)],
        ],
    )
    def run(idx_hbm, w_hbm, out_hbm, idx_v, out_v, bufs, sems):
        wid = lax.axis_index("s") * _NUM_CORES + lax.axis_index("c")
        pltpu.sync_copy(idx_hbm.at[pl.ds(wid * nch, nch)], idx_v)

        def fire(s, b):
            return [
                pltpu.async_copy(
                    w_hbm.at[idx_v.at[s * _KB + k]],
                    bufs[b].at[pl.ds(k * cl, cl)],
                    sems[b],
                )
                for k in range(_KB)
            ]

        cps = [fire(s, s % _NBUF) for s in range(_NBUF)]

        for s in range(ns):
            b = s % _NBUF
            for cp in cps[b]:
                cp.wait()
            buf = bufs[b]

            @plsc.parallel_loop(0, out_ps, 1)
            def body(o, buf=buf, s=s):
                m = o * F
                for h in range(0, D, _LANES):
                    pa = buf[m, pl.ds(h, _LANES)]
                    pb = buf[m + 1, pl.ds(h, _LANES)]
                    for j in range(2, F, 2):
                        pa = pa + buf[m + j, pl.ds(h, _LANES)]
                        pb = pb + buf[m + j + 1, pl.ds(h, _LANES)]
                    out_v[s * out_ps + o, pl.ds(h, _LANES)] = pa + pb

            if s + _NBUF < ns:
                cps[b] = fire(s + _NBUF, b)

        pltpu.sync_copy(out_v, out_hbm.at[pl.ds(wid * nb, nb)])

    return run(idx2, weight)


def kernel(input, weight):
    B, F = input.shape
    _, D = weight.shape
    rpc = 128 // F
    idx2 = input.reshape(B // rpc, rpc * F)
    return _embedding_bag(idx2, weight, B=B, F=F, D=D)


# speedup vs baseline: 2.3202x; 1.1630x over previous
import functools

import jax
import jax.numpy as jnp
from jax import lax
from jax.experimental import pallas as pl
from jax.experimental.pallas import tpu as pltpu
from jax.experimental.pallas import tpu_sc as plsc

_NUM_CORES = 2
_NUM_SUBCORES = 16
_NW = _NUM_CORES * _NUM_SUBCORES
_LANES = 16
_NBUF = 3
_KB = 8


def _mesh():
    return plsc.VectorSubcoreMesh(core_axis_name="c", subcore_axis_name="s")


@functools.partial(jax.jit, static_argnames=("V", "D"))
def _pack_table(wt, *, V, D):
    ng = (V + 127) // 128
    dg = 128 * D // 128
    gpw = (ng + _NW - 1) // _NW
    nit = (gpw + 1) // 2
    nb_blocks = D // 8

    @functools.partial(
        pl.kernel,
        out_type=jax.ShapeDtypeStruct((ng * dg, 128), jnp.float32),
        mesh=_mesh(),
        compiler_params=pltpu.CompilerParams(
            use_tc_tiling_on_sc=True,
            needs_layout_passes=False,
            disable_bounds_checks=True,
        ),
        scratch_types=[
            [pltpu.VMEM((D, 128), jnp.float32) for _ in range(2)],
            [pltpu.VMEM((dg, 128), jnp.float32) for _ in range(2)],
            [pltpu.SemaphoreType.DMA for _ in range(2)],
            [pltpu.SemaphoreType.DMA for _ in range(2)],
        ],
    )
    def run(wt_hbm, wr_hbm, tbufs, obufs, isems, osems):
        wid = lax.axis_index("s") * _NUM_CORES + lax.axis_index("c")
        base = wid * gpw
        lim = jnp.minimum(base + gpw, ng)
        lane = jax.lax.iota(jnp.int32, 16)

        def fire_in(g, t):
            @pl.when(g < lim)
            def _():
                for b in range(nb_blocks):
                    pltpu.async_copy(
                        wt_hbm.at[pl.ds(b * 8, 8), pl.ds(g * 128, 128)],
                        tbufs[t].at[pl.ds(b * 8, 8)],
                        isems[t],
                    )

        def drain(sem, ref):
            pltpu.make_async_copy(wr_hbm.at[pl.ds(0, ref.shape[0])], ref, sem).wait()

        def process(g, t, it):
            @pl.when(g < lim)
            def _():
                drain(isems[t], tbufs[t])

                @pl.when(it > 0)
                def _():
                    drain(osems[t], obufs[t])

                tb, ob = tbufs[t], obufs[t]

                @plsc.parallel_loop(0, dg, 1)
                def _(r):
                    for q in range(4):
                        col = 4 * r + q
                        for h in range(2):
                            v = plsc.load_gather(
                                tb, [h * 16 + lane, col + lane * 0]
                            )
                            ob[r, pl.ds(q * 32 + h * 16, 16)] = v

                pltpu.async_copy(
                    obufs[t], wr_hbm.at[pl.ds(g * dg, dg)], osems[t]
                )

        fire_in(base, 0)

        def body(it, carry):
            g0 = base + 2 * it
            fire_in(g0 + 1, 1)
            process(g0, 0, it)
            fire_in(g0 + 2, 0)
            process(g0 + 1, 1, it)
            return carry

        lax.fori_loop(0, nit, body, 0)
        drain(osems[0], obufs[0])
        drain(osems[1], obufs[1])

    return run(wt)


@functools.partial(jax.jit, static_argnames=("B", "F", "D"))
def _embedding_bag(idx2, weight, *, B, F, D):
    nb = B // _NW
    rpc = 128 // F
    cl = rpc * F
    nch = nb // rpc
    ns = nch // _KB
    rows_ps = _KB * cl
    out_ps = _KB * rpc

    @functools.partial(
        pl.kernel,
        out_type=jax.ShapeDtypeStruct((B, D), jnp.float32),
        mesh=_mesh(),
        compiler_params=pltpu.CompilerParams(use_tc_tiling_on_sc=False),
        scratch_types=[
            pltpu.VMEM((nch, cl), jnp.int32),
            pltpu.VMEM((nb, D), jnp.float32),
            [pltpu.VMEM((rows_ps, D), jnp.float32) for _ in range(_NBUF)],
            [pltpu.SemaphoreType.DMA for _ in range(_NBUF)],
        ],
    )
    def run(idx_hbm, w_hbm, out_hbm, idx_v, out_v, bufs, sems):
        wid = lax.axis_index("s") * _NUM_CORES + lax.axis_index("c")
        pltpu.sync_copy(idx_hbm.at[pl.ds(wid * nch, nch)], idx_v)

        def fire(s, b):
            return [
                pltpu.async_copy(
                    w_hbm.at[idx_v.at[s * _KB + k]],
                    bufs[b].at[pl.ds(k * cl, cl)],
                    sems[b],
                )
                for k in range(_KB)
            ]

        cps = [fire(s, s % _NBUF) for s in range(_NBUF)]

        for s in range(ns):
            b = s % _NBUF
            for cp in cps[b]:
                cp.wait()
            buf = bufs[b]

            @plsc.parallel_loop(0, out_ps, 1)
            def body(o, buf=buf, s=s):
                m = o * F
                for h in range(0, D, _LANES):
                    pa = buf[m, pl.ds(h, _LANES)]
                    pb = buf[m + 1, pl.ds(h, _LANES)]
                    for j in range(2, F, 2):
                        pa = pa + buf[m + j, pl.ds(h, _LANES)]
                        pb = pb + buf[m + j + 1, pl.ds(h, _LANES)]
                    out_v[s * out_ps + o, pl.ds(h, _LANES)] = pa + pb

            nxt = s + _NBUF
            if nxt < ns:
                cps[b] = fire(nxt, b)

        pltpu.sync_copy(out_v, out_hbm.at[pl.ds(wid * nb, nb)])

    return run(idx2, weight)


def kernel(input, weight):
    B, F = input.shape
    V1, D = weight.shape
    rpc = 128 // F
    idx2 = input.reshape(B // rpc, rpc * F)
    w128 = _pack_table(weight.T, V=V1, D=D)
    w2 = w128.reshape(-1, D)
    return _embedding_bag(idx2, w2, B=B, F=F, D=D)


# speedup vs baseline: 2.7903x; 1.2026x over previous
import functools

import jax
import jax.numpy as jnp
from jax import lax
from jax.experimental import pallas as pl
from jax.experimental.pallas import tpu as pltpu
from jax.experimental.pallas import tpu_sc as plsc

_NUM_CORES = 2
_NUM_SUBCORES = 16
_NW = _NUM_CORES * _NUM_SUBCORES
_LANES = 16
_NBUF = 3
_KB = 8
_X = 512


@functools.partial(jax.jit, static_argnames=("V", "D"))
def _pack_table(wt, *, V, D):
    M = ((V + 3) // 4 + _X - 1) // _X * _X
    ng = M // _X
    lb = (V - 1) // _X

    def body(r0, r1, r2, r3, o_ref):
        x = jnp.concatenate([r0[...], r1[...], r2[...], r3[...]], axis=0)
        o_ref[...] = x.T

    f = pl.pallas_call(
        body,
        grid=(ng,),
        in_specs=[
            pl.BlockSpec(
                (D, _X), lambda i, a=a: (0, jnp.minimum(a * ng + i, lb))
            )
            for a in range(4)
        ],
        out_specs=pl.BlockSpec((_X, 128), lambda i: (i, 0)),
        out_shape=jax.ShapeDtypeStruct((M, 128), jnp.float32),
    )
    return f(wt, wt, wt, wt)


@functools.partial(jax.jit, static_argnames=("B", "F", "D"))
def _embedding_bag(idx2, weight, *, B, F, D):
    nb = B // _NW
    rpc = 128 // F
    cl = rpc * F
    nch = nb // rpc
    ns = nch // _KB
    rows_ps = _KB * cl
    out_ps = _KB * rpc

    @functools.partial(
        pl.kernel,
        out_type=jax.ShapeDtypeStruct((B, D), jnp.float32),
        mesh=plsc.VectorSubcoreMesh(core_axis_name="c", subcore_axis_name="s"),
        compiler_params=pltpu.CompilerParams(use_tc_tiling_on_sc=False),
        scratch_types=[
            pltpu.VMEM((nch, cl), jnp.int32),
            pltpu.VMEM((nb, D), jnp.float32),
            [pltpu.VMEM((rows_ps, D), jnp.float32) for _ in range(_NBUF)],
            [pltpu.SemaphoreType.DMA for _ in range(_NBUF)],
        ],
    )
    def run(idx_hbm, w_hbm, out_hbm, idx_v, out_v, bufs, sems):
        wid = lax.axis_index("s") * _NUM_CORES + lax.axis_index("c")
        pltpu.sync_copy(idx_hbm.at[pl.ds(wid * nch, nch)], idx_v)

        def fire(s, b):
            return [
                pltpu.async_copy(
                    w_hbm.at[idx_v.at[s * _KB + k]],
                    bufs[b].at[pl.ds(k * cl, cl)],
                    sems[b],
                )
                for k in range(_KB)
            ]

        cps = [fire(s, s % _NBUF) for s in range(_NBUF)]

        for s in range(ns):
            b = s % _NBUF
            for cp in cps[b]:
                cp.wait()
            buf = bufs[b]

            @plsc.parallel_loop(0, out_ps, 1)
            def body(o, buf=buf, s=s):
                m = o * F
                for h in range(0, D, _LANES):
                    pa = buf[m, pl.ds(h, _LANES)]
                    pb = buf[m + 1, pl.ds(h, _LANES)]
                    for j in range(2, F, 2):
                        pa = pa + buf[m + j, pl.ds(h, _LANES)]
                        pb = pb + buf[m + j + 1, pl.ds(h, _LANES)]
                    out_v[s * out_ps + o, pl.ds(h, _LANES)] = pa + pb

            nxt = s + _NBUF
            if nxt < ns:
                cps[b] = fire(nxt, b)

        pltpu.sync_copy(out_v, out_hbm.at[pl.ds(wid * nb, nb)])

    return run(idx2, weight)


def kernel(input, weight):
    B, F = input.shape
    V1, D = weight.shape
    rpc = 128 // F
    M = ((V1 + 3) // 4 + _X - 1) // _X * _X
    w128 = _pack_table(weight.T, V=V1, D=D)
    w2 = w128.reshape(-1, D)
    idx2 = input.reshape(B // rpc, rpc * F)
    a = (idx2 // M).astype(jnp.int32)
    idxL = 4 * (idx2 - a * M) + a
    return _embedding_bag(idxL, w2, B=B, F=F, D=D)
